# baseline (speedup 1.0000x reference)
import functools

import jax
import jax.numpy as jnp
from jax import lax
from jax.experimental import pallas as pl
from jax.experimental.pallas import tpu as pltpu
from jax.experimental.pallas import tpu_sc as plsc

B = 16384
F = 26
FIELD_VOCAB = 40000
D = 16
HID = 256
TOT = B * F

NC = 2
NS = 16
NW = NC * NS
PER_W = TOT // NW
N_BLK = 4
BLK = PER_W // N_BLK
IDX_PER_DMA = 128
DMAS = BLK // IDX_PER_DMA
VPG = 13
N_GRP = BLK // (VPG * 16)


def _sc_body(x_hbm, table_hbm, out_hbm, idx_v, rows_v, sem):
    wid = lax.axis_index("s") * NC + lax.axis_index("c")
    base = wid * PER_W
    lane = lax.iota(jnp.int32, 16)
    offs = [((lane + r * 16) % F) * FIELD_VOCAB for r in range(VPG)]

    for blk in range(N_BLK):
        eb = base + blk * BLK
        pltpu.sync_copy(x_hbm.at[pl.ds(eb, BLK)], idx_v)

        def grp(g, _):
            gb = g * (VPG * 16)
            for r in range(VPG):
                sl = pl.ds(gb + r * 16, 16)
                idx_v[sl] = idx_v[sl] + offs[r]
            return 0

        lax.fori_loop(0, N_GRP, grp, 0)

        copies = []
        for j in range(DMAS):
            sl = pl.ds(j * IDX_PER_DMA, IDX_PER_DMA)
            copies.append(
                pltpu.async_copy(table_hbm.at[idx_v.at[sl]], rows_v.at[sl], sem)
            )
        for c in copies:
            c.wait()

        pltpu.sync_copy(rows_v, out_hbm.at[pl.ds(eb, BLK)])


_sc_gather = functools.partial(
    pl.kernel,
    mesh=plsc.VectorSubcoreMesh(core_axis_name="c", subcore_axis_name="s"),
    compiler_params=pltpu.CompilerParams(use_tc_tiling_on_sc=False),
    out_type=jax.ShapeDtypeStruct((TOT, D), jnp.float32),
    scratch_types=[
        pltpu.VMEM((BLK,), jnp.int32),
        pltpu.VMEM((BLK, D), jnp.float32),
        pltpu.SemaphoreType.DMA,
    ],
)(_sc_body)


BM = 2048
N_MLP = B // BM


def _mlp_body(flat_hbm, w1, b1, w2t, b2, ob, xb, sem):
    i = pl.program_id(0)
    slot = lax.rem(i, 2)
    nslot = lax.rem(i + 1, 2)

    @pl.when(i == 0)
    def _():
        pltpu.make_async_copy(
            flat_hbm.at[pl.ds(0, BM)], xb.at[0], sem.at[0]
        ).start()

    @pl.when(i + 1 < N_MLP)
    def _():
        pltpu.make_async_copy(
            flat_hbm.at[pl.ds((i + 1) * BM, BM)], xb.at[nslot], sem.at[nslot]
        ).start()

    pltpu.make_async_copy(
        flat_hbm.at[pl.ds(i * BM, BM)], xb.at[slot], sem.at[slot]
    ).wait()

    h = jnp.dot(xb[slot], w1[...], preferred_element_type=jnp.float32)
    h = jnp.maximum(h + b1[...], 0.0)
    ob[...] = jnp.sum(h * w2t[...], axis=1, keepdims=True) + b2[...]


_mlp = pl.pallas_call(
    _mlp_body,
    grid=(N_MLP,),
    in_specs=[
        pl.BlockSpec(memory_space=pltpu.MemorySpace.HBM),
        pl.BlockSpec((F * D, HID), lambda i: (0, 0)),
        pl.BlockSpec((1, HID), lambda i: (0, 0)),
        pl.BlockSpec((1, HID), lambda i: (0, 0)),
        pl.BlockSpec((1, 1), lambda i: (0, 0)),
    ],
    out_specs=pl.BlockSpec((BM, 1), lambda i: (i, 0)),
    out_shape=jax.ShapeDtypeStruct((B, 1), jnp.float32),
    scratch_shapes=[
        pltpu.VMEM((2, BM, F * D), jnp.float32),
        pltpu.SemaphoreType.DMA((2,)),
    ],
)


def kernel(x, table, W1, b1, W2, b2):
    rows = _sc_gather(x.reshape(-1), table)
    flat = rows.reshape(B, F * D)
    return _mlp(flat, W1, b1.reshape(1, HID), W2.reshape(1, HID),
                b2.reshape(1, 1))


# speedup vs baseline: 1.2647x; 1.2647x over previous
import functools

import jax
import jax.numpy as jnp
from jax import lax
from jax.experimental import pallas as pl
from jax.experimental.pallas import tpu as pltpu
from jax.experimental.pallas import tpu_sc as plsc

B = 16384
F = 26
FIELD_VOCAB = 40000
D = 16
HID = 256
TOT = B * F

NC = 2
NS = 16
NW = NC * NS
PER_W = TOT // NW
N_BLK = 4
BLK = PER_W // N_BLK
IDX_PER_DMA = 128
DMAS = BLK // IDX_PER_DMA
VPG = 13
N_GRP = BLK // (VPG * 16)


def _sc_body(x_hbm, table_hbm, out_hbm, idx_v, rows_v, sem):
    wid = lax.axis_index("s") * NC + lax.axis_index("c")
    base = wid * PER_W
    lane = lax.iota(jnp.int32, 16)
    offs = [((lane + r * 16) % F) * FIELD_VOCAB for r in range(VPG)]

    for blk in range(N_BLK):
        eb = base + blk * BLK
        pltpu.sync_copy(x_hbm.at[pl.ds(eb, BLK)], idx_v)

        def grp(g, _):
            gb = g * (VPG * 16)
            for r in range(VPG):
                sl = pl.ds(gb + r * 16, 16)
                idx_v[sl] = idx_v[sl] + offs[r]
            return 0

        lax.fori_loop(0, N_GRP, grp, 0)

        copies = []
        for j in range(DMAS):
            sl = pl.ds(j * IDX_PER_DMA, IDX_PER_DMA)
            copies.append(
                pltpu.async_copy(table_hbm.at[idx_v.at[sl]], rows_v.at[sl], sem)
            )
        for c in copies:
            c.wait()

        pltpu.sync_copy(rows_v, out_hbm.at[pl.ds(eb, BLK)])


_sc_gather = functools.partial(
    pl.kernel,
    mesh=plsc.VectorSubcoreMesh(core_axis_name="c", subcore_axis_name="s"),
    compiler_params=pltpu.CompilerParams(use_tc_tiling_on_sc=False),
    out_type=jax.ShapeDtypeStruct((TOT, D), jnp.float32),
    scratch_types=[
        pltpu.VMEM((BLK,), jnp.int32),
        pltpu.VMEM((BLK, D), jnp.float32),
        pltpu.SemaphoreType.DMA,
    ],
)(_sc_body)


N_ROWS = 1040000
FMT_BLK = 41600
N_FMT = N_ROWS // FMT_BLK
FMT_RB = FMT_BLK // 8


def _fmt_body(tb, ob):
    t = tb[...].T
    t3 = t.reshape(FMT_RB, 8, D)
    ob[...] = jnp.concatenate([t3[:, s, :] for s in range(8)], axis=1)


_fmt = pl.pallas_call(
    _fmt_body,
    grid=(N_FMT,),
    in_specs=[pl.BlockSpec((D, FMT_BLK), lambda i: (0, i))],
    out_specs=pl.BlockSpec((FMT_RB, 128), lambda i: (i, 0)),
    out_shape=jax.ShapeDtypeStruct((N_ROWS * D // 128, 128), jnp.float32),
    compiler_params=pltpu.CompilerParams(
        dimension_semantics=("parallel",),
    ),
)


BM = 2048
N_MLP = B // BM


def _mlp_body(flat_hbm, w1, b1, w2t, b2, ob, xb, sem):
    i = pl.program_id(0)
    slot = lax.rem(i, 2)
    nslot = lax.rem(i + 1, 2)

    @pl.when(i == 0)
    def _():
        pltpu.make_async_copy(
            flat_hbm.at[pl.ds(0, BM)], xb.at[0], sem.at[0]
        ).start()

    @pl.when(i + 1 < N_MLP)
    def _():
        pltpu.make_async_copy(
            flat_hbm.at[pl.ds((i + 1) * BM, BM)], xb.at[nslot], sem.at[nslot]
        ).start()

    pltpu.make_async_copy(
        flat_hbm.at[pl.ds(i * BM, BM)], xb.at[slot], sem.at[slot]
    ).wait()

    h = jnp.dot(xb[slot], w1[...], preferred_element_type=jnp.float32)
    h = jnp.maximum(h + b1[...], 0.0)
    ob[...] = jnp.sum(h * w2t[...], axis=1, keepdims=True) + b2[...]


_mlp = pl.pallas_call(
    _mlp_body,
    grid=(N_MLP,),
    in_specs=[
        pl.BlockSpec(memory_space=pltpu.MemorySpace.HBM),

        pl.BlockSpec((F * D, HID), lambda i: (0, 0)),
        pl.BlockSpec((1, HID), lambda i: (0, 0)),
        pl.BlockSpec((1, HID), lambda i: (0, 0)),
        pl.BlockSpec((1, 1), lambda i: (0, 0)),
    ],
    out_specs=pl.BlockSpec((BM, 1), lambda i: (i, 0)),
    out_shape=jax.ShapeDtypeStruct((B, 1), jnp.float32),
    scratch_shapes=[
        pltpu.VMEM((2, BM, F * D), jnp.float32),
        pltpu.SemaphoreType.DMA((2,)),
    ],
)


def kernel(x, table, W1, b1, W2, b2):
    table_rm = _fmt(table.T).reshape(N_ROWS, D)
    rows = _sc_gather(x.reshape(-1), table_rm)
    flat = rows.reshape(B, F * D)
    return _mlp(flat, W1, b1.reshape(1, HID), W2.reshape(1, HID),
                b2.reshape(1, 1))


# speedup vs baseline: 1.3441x; 1.0628x over previous
import functools

import jax
import jax.numpy as jnp
from jax import lax
from jax.experimental import pallas as pl
from jax.experimental.pallas import tpu as pltpu
from jax.experimental.pallas import tpu_sc as plsc

B = 16384
F = 26
FIELD_VOCAB = 40000
D = 16
HID = 256
TOT = B * F

NC = 2
NS = 16
NW = NC * NS
PER_W = TOT // NW
N_BLK = 4
BLK = PER_W // N_BLK
IDX_PER_DMA = 128
DMAS = BLK // IDX_PER_DMA
VPG = 13
N_GRP = BLK // (VPG * 16)


def _sc_body(x_hbm, table_hbm, out_hbm, idx_v, rows_v, sem):
    wid = lax.axis_index("s") * NC + lax.axis_index("c")
    base = wid * PER_W
    lane = lax.iota(jnp.int32, 16)
    offs = [((lane + r * 16) % F) * FIELD_VOCAB for r in range(VPG)]

    for blk in range(N_BLK):
        eb = base + blk * BLK
        pltpu.sync_copy(x_hbm.at[pl.ds(eb, BLK)], idx_v)

        def grp(g, _):
            gb = g * (VPG * 16)
            for r in range(VPG):
                sl = pl.ds(gb + r * 16, 16)
                idx_v[sl] = idx_v[sl] + offs[r]
            return 0

        lax.fori_loop(0, N_GRP, grp, 0)

        copies = []
        for j in range(DMAS):
            sl = pl.ds(j * IDX_PER_DMA, IDX_PER_DMA)
            copies.append(
                pltpu.async_copy(table_hbm.at[idx_v.at[sl]], rows_v.at[sl], sem)
            )
        for c in copies:
            c.wait()

        pltpu.sync_copy(rows_v, out_hbm.at[pl.ds(eb, BLK)])


_sc_gather = functools.partial(
    pl.kernel,
    mesh=plsc.VectorSubcoreMesh(core_axis_name="c", subcore_axis_name="s"),
    compiler_params=pltpu.CompilerParams(use_tc_tiling_on_sc=False),
    out_type=jax.ShapeDtypeStruct((TOT, D), jnp.float32),
    scratch_types=[
        pltpu.VMEM((BLK,), jnp.int32),
        pltpu.VMEM((BLK, D), jnp.float32),
        pltpu.SemaphoreType.DMA,
    ],
)(_sc_body)


N_ROWS = 1040000
FMT_BLK = 41600
N_FMT = N_ROWS // FMT_BLK
FMT_RB = FMT_BLK // 8


def _fmt_body(tb, ob):
    eye = jnp.eye(D, dtype=jnp.float32)
    t = jax.lax.dot_general(
        tb[...], eye, (((0,), (0,)), ((), ())),
        preferred_element_type=jnp.float32,
    )
    t3 = t.reshape(FMT_RB, 8, D)
    ob[...] = jnp.concatenate([t3[:, s, :] for s in range(8)], axis=1)


_fmt = pl.pallas_call(
    _fmt_body,
    grid=(N_FMT,),
    in_specs=[pl.BlockSpec((D, FMT_BLK), lambda i: (0, i))],
    out_specs=pl.BlockSpec((FMT_RB, 128), lambda i: (i, 0)),
    out_shape=jax.ShapeDtypeStruct((N_ROWS * D // 128, 128), jnp.float32),
    compiler_params=pltpu.CompilerParams(
        dimension_semantics=("parallel",),
    ),
)


BM = 2048
N_MLP = B // BM


def _mlp_body(flat_hbm, w1, b1, w2t, b2, ob, xb, sem):
    i = pl.program_id(0)
    slot = lax.rem(i, 2)
    nslot = lax.rem(i + 1, 2)

    @pl.when(i == 0)
    def _():
        pltpu.make_async_copy(
            flat_hbm.at[pl.ds(0, BM)], xb.at[0], sem.at[0]
        ).start()

    @pl.when(i + 1 < N_MLP)
    def _():
        pltpu.make_async_copy(
            flat_hbm.at[pl.ds((i + 1) * BM, BM)], xb.at[nslot], sem.at[nslot]
        ).start()

    pltpu.make_async_copy(
        flat_hbm.at[pl.ds(i * BM, BM)], xb.at[slot], sem.at[slot]
    ).wait()

    h = jnp.dot(xb[slot], w1[...], preferred_element_type=jnp.float32)
    h = jnp.maximum(h + b1[...], 0.0)
    ob[...] = jnp.sum(h * w2t[...], axis=1, keepdims=True) + b2[...]


_mlp = pl.pallas_call(
    _mlp_body,
    grid=(N_MLP,),
    in_specs=[
        pl.BlockSpec(memory_space=pltpu.MemorySpace.HBM),

        pl.BlockSpec((F * D, HID), lambda i: (0, 0)),
        pl.BlockSpec((1, HID), lambda i: (0, 0)),
        pl.BlockSpec((1, HID), lambda i: (0, 0)),
        pl.BlockSpec((1, 1), lambda i: (0, 0)),
    ],
    out_specs=pl.BlockSpec((BM, 1), lambda i: (i, 0)),
    out_shape=jax.ShapeDtypeStruct((B, 1), jnp.float32),
    scratch_shapes=[
        pltpu.VMEM((2, BM, F * D), jnp.float32),
        pltpu.SemaphoreType.DMA((2,)),
    ],
)


def kernel(x, table, W1, b1, W2, b2):
    table_rm = _fmt(table.T).reshape(N_ROWS, D)
    rows = _sc_gather(x.reshape(-1), table_rm)
    flat = rows.reshape(B, F * D)
    return _mlp(flat, W1, b1.reshape(1, HID), W2.reshape(1, HID),
                b2.reshape(1, 1))
